# baseline (speedup 1.0000x reference)
import functools

import jax
import jax.numpy as jnp
from jax import lax
from jax.experimental import pallas as pl
from jax.experimental.pallas import tpu as pltpu
from jax.experimental.pallas import tpu_sc as plsc

N = 10000
E = 320000
D = 128
DE = 16
G = 64

NC = 2
NS = 16
NW = NC * NS
EPW = E // NW
C = 64
NCHUNK = EPW // C
CTAIL = EPW - NCHUNK * C
RPS = 624
NTAIL = N - RPS * NS


def _edge_sc_body(h_hbm, ea_hbm, src_hbm, dst_hbm, out_hbm,
                  sidx0, sidx1, didx0, didx1, didx2,
                  hr0, hr1, hr2, ea0, ea1,
                  sidxt, didxt, hrt, eat, aggr,
                  sems0, sems1, semd0, semd1, semd2,
                  semg0, semg1, semg2, semsc0, semsc1, semsc2,
                  seme0, seme1):
    c = lax.axis_index("c")
    s = lax.axis_index("s")
    wid = s * NC + c
    base = wid * EPW
    base2 = wid * (EPW // 2)
    sidxs = (sidx0, sidx1)
    semss = (sems0, sems1)
    didxs = (didx0, didx1, didx2)
    semds = (semd0, semd1, semd2)
    hrs = (hr0, hr1, hr2)
    semgs = (semg0, semg1, semg2)
    semscs = (semsc0, semsc1, semsc2)
    eabufs = (ea0, ea1)
    semes = (seme0, seme1)

    def load_sidx_ea(j, u):
        pltpu.async_copy(src_hbm.at[pl.ds(base + j * C, C)], sidxs[u % 2],
                         semss[u % 2])
        pltpu.async_copy(ea_hbm.at[pl.ds(base2 + j * (C // 2), C // 2)],
                         eabufs[u % 2], semes[u % 2])

    def load_didx(j, u):
        pltpu.async_copy(dst_hbm.at[pl.ds(base + j * C, C)], didxs[u % 3],
                         semds[u % 3])

    def start_gather(j, u):
        pltpu.make_async_copy(src_hbm.at[pl.ds(base + j * C, C)],
                              sidxs[u % 2], semss[u % 2]).wait()
        pltpu.async_copy(h_hbm.at[sidxs[u % 2]], hrs[u % 3], semgs[u % 3])

    def wait_scatter(u):
        pltpu.make_async_copy(hrs[u % 3], aggr.at[didxs[u % 3]],
                              semscs[u % 3]).wait()

    load_sidx_ea(0, 0)
    load_sidx_ea(1, 1)
    load_didx(0, 0)
    start_gather(0, 0)

    pltpu.sync_copy(h_hbm.at[pl.ds(s * RPS, RPS)], aggr.at[pl.ds(s * RPS, RPS)])

    @pl.when(s == NS - 1)
    def _init_tail():
        pltpu.sync_copy(h_hbm.at[pl.ds(RPS * NS, NTAIL)],
                        aggr.at[pl.ds(RPS * NS, NTAIL)])

    plsc.subcore_barrier()

    def process(j, u, guard_lo=False, last=False, skip_prefetch=False):
        hrows = hrs[u % 3]
        eabuf = eabufs[u % 2]
        didx = didxs[u % 3]

        if not last:
            if guard_lo:
                @pl.when(j >= 2)
                def _w():
                    wait_scatter(u + 1)
            else:
                wait_scatter(u + 1)
            load_didx(j + 1, u + 1)
            start_gather(j + 1, u + 1)

        pltpu.make_async_copy(ea_hbm.at[pl.ds(base2 + j * (C // 2), C // 2)],
                              eabuf, semes[u % 2]).wait()
        pltpu.make_async_copy(h_hbm.at[sidxs[u % 2]], hrows,
                              semgs[u % 3]).wait()

        def row(q, carry2):
            for rr in range(2):
                r = q * 2 + rr
                for k in range(D // 32):
                    v = eabuf[q, pl.ds(rr * 64 + k * 16, 16)]
                    ea_lo = lax.bitcast_convert_type(v << 16, jnp.float32)
                    ea_hi = lax.bitcast_convert_type(
                        v & jnp.int32(-65536), jnp.float32)
                    sl_lo = pl.ds(k * 32, 16)
                    sl_hi = pl.ds(k * 32 + 16, 16)
                    hrows[r, sl_lo] = jnp.maximum(hrows[r, sl_lo] + ea_lo,
                                                  0.0)
                    hrows[r, sl_hi] = jnp.maximum(hrows[r, sl_hi] + ea_hi,
                                                  0.0)
            return carry2

        lax.fori_loop(0, C // 2, row, 0)
        pltpu.make_async_copy(dst_hbm.at[pl.ds(base + j * C, C)], didx,
                              semds[u % 3]).wait()
        pltpu.async_copy(hrows, aggr.at[didx], semscs[u % 3], add=True)
        if not last and not skip_prefetch:
            load_sidx_ea(j + 2, u + 2)

    UNROLL = 6
    NMAIN = (NCHUNK - 6) // UNROLL * UNROLL

    def block(t, carry):
        for u in range(UNROLL):
            process(t * UNROLL + u, u, guard_lo=(u < 2))
        return carry

    lax.fori_loop(0, NMAIN // UNROLL, block, 0)
    for j in range(NMAIN, NCHUNK):
        process(j, j % UNROLL, last=(j + 1 >= NCHUNK),
                skip_prefetch=(j + 2 >= NCHUNK))
    for jj in (NCHUNK - 3, NCHUNK - 2, NCHUNK - 1):
        wait_scatter(jj)

    toff = base + NCHUNK * C
    pltpu.sync_copy(src_hbm.at[pl.ds(toff, CTAIL)], sidxt)
    pltpu.sync_copy(dst_hbm.at[pl.ds(toff, CTAIL)], didxt)
    pltpu.sync_copy(ea_hbm.at[pl.ds(base2 + NCHUNK * (C // 2), CTAIL // 2)],
                    eat)
    pltpu.async_copy(h_hbm.at[sidxt], hrt, semg0).wait()

    def trow(q, carry2):
        for rr in range(2):
            r = q * 2 + rr
            for k in range(D // 32):
                v = eat[q, pl.ds(rr * 64 + k * 16, 16)]
                ea_lo = lax.bitcast_convert_type(v << 16, jnp.float32)
                ea_hi = lax.bitcast_convert_type(v & jnp.int32(-65536),
                                                 jnp.float32)
                sl_lo = pl.ds(k * 32, 16)
                sl_hi = pl.ds(k * 32 + 16, 16)
                hrt[r, sl_lo] = jnp.maximum(hrt[r, sl_lo] + ea_lo, 0.0)
                hrt[r, sl_hi] = jnp.maximum(hrt[r, sl_hi] + ea_hi, 0.0)
        return carry2

    lax.fori_loop(0, CTAIL // 2, trow, 0)
    pltpu.sync_copy(hrt, aggr.at[didxt], add=True)

    plsc.subcore_barrier()
    pltpu.sync_copy(aggr.at[pl.ds(s * RPS, RPS)],
                    out_hbm.at[c, pl.ds(s * RPS, RPS)])

    @pl.when(s == NS - 1)
    def _out_tail():
        pltpu.sync_copy(aggr.at[pl.ds(RPS * NS, NTAIL)],
                        out_hbm.at[c, pl.ds(RPS * NS, NTAIL)])


@jax.jit
def _edge_sc(h, ea, src, dst):
    mesh = plsc.VectorSubcoreMesh(core_axis_name="c", subcore_axis_name="s")
    f = pl.kernel(
        _edge_sc_body,
        out_type=jax.ShapeDtypeStruct((NC, N, D), jnp.float32),
        mesh=mesh,
        scratch_types=(
            [pltpu.VMEM((C,), jnp.int32)] * 5
            + [pltpu.VMEM((C, D), jnp.float32)] * 3
            + [pltpu.VMEM((C // 2, D), jnp.int32)] * 2
            + [pltpu.VMEM((CTAIL,), jnp.int32)] * 2
            + [pltpu.VMEM((CTAIL, D), jnp.float32)]
            + [pltpu.VMEM((CTAIL // 2, D), jnp.int32)]
            + [pltpu.VMEM_SHARED((N, D), jnp.float32)]
            + [pltpu.SemaphoreType.DMA] * 13
        ),
    )
    return f(h, ea, src, dst)


def _ea_matmul_body(a_ref, w_ref, b_ref, o_ref):
    ea = (jnp.dot(a_ref[...], w_ref[...], preferred_element_type=jnp.float32)
          + b_ref[...])
    o_ref[...] = ea.astype(jnp.bfloat16)


@jax.jit
def _ea_matmul(edge_attr, W, b):
    BE = 4000
    return pl.pallas_call(
        _ea_matmul_body,
        grid=(E // BE,),
        in_specs=[
            pl.BlockSpec((BE, DE), lambda e: (e, 0)),
            pl.BlockSpec((DE, D), lambda e: (0, 0)),
            pl.BlockSpec((1, D), lambda e: (0, 0)),
        ],
        out_specs=pl.BlockSpec((BE, D), lambda e: (e, 0)),
        out_shape=jax.ShapeDtypeStruct((E, D), jnp.bfloat16),
    )(edge_attr, W, b.reshape(1, D))


def _mlp_body(h_ref, p_ref, w1_ref, b1_ref, g_ref, be_ref, w2_ref, b2_ref,
              o_ref):
    z = p_ref[0] + p_ref[1] - h_ref[...]
    z1 = jnp.dot(z, w1_ref[...], preferred_element_type=jnp.float32) + b1_ref[...]
    mu = jnp.mean(z1, axis=0, keepdims=True)
    zc = z1 - mu
    var = jnp.mean(zc * zc, axis=0, keepdims=True)
    zn = zc * (g_ref[...] * lax.rsqrt(var + 1e-5)) + be_ref[...]
    z2 = jnp.maximum(zn, 0.0)
    z3 = jnp.dot(z2, w2_ref[...], preferred_element_type=jnp.float32) + b2_ref[...]
    o_ref[...] = jnp.maximum(z3, 0.0)


@jax.jit
def _mlp(h, p, W1, b1, gamma, beta, W2, b2):
    return pl.pallas_call(
        _mlp_body,
        out_shape=jax.ShapeDtypeStruct((N, D), jnp.float32),
    )(h, p, W1, b1.reshape(1, D), gamma.reshape(1, D), beta.reshape(1, D),
      W2, b2.reshape(1, D))


def _readout_body(h_ref, b_ref, wo1_ref, bo1_ref, wo2_ref, bo2_ref, o_ref):
    bvec = b_ref[...]
    gids = lax.broadcasted_iota(jnp.int32, (G, N), 0)
    onehot = (gids == bvec).astype(jnp.float32)
    sums = jnp.dot(onehot, h_ref[...], preferred_element_type=jnp.float32)
    cnt = jnp.sum(onehot, axis=1, keepdims=True)
    pooled = sums / jnp.maximum(cnt, 1.0)
    t = jnp.maximum(
        jnp.dot(pooled, wo1_ref[...], preferred_element_type=jnp.float32)
        + bo1_ref[...], 0.0)
    o_ref[...] = (
        jnp.dot(t, wo2_ref[...], preferred_element_type=jnp.float32)
        + bo2_ref[...]
    )


@jax.jit
def _readout(h, batch, Wo1, bo1, Wo2, bo2):
    return pl.pallas_call(
        _readout_body,
        out_shape=jax.ShapeDtypeStruct((G, D), jnp.float32),
    )(h, batch.reshape(1, N), Wo1, bo1.reshape(1, D), Wo2, bo2.reshape(1, D))


def kernel(x, edge_index, edge_attr, batch, edge_W, edge_b, W1, b1, gamma,
           beta, W2, b2, Wo1, bo1, Wo2, bo2):
    src = edge_index[0]
    dst = edge_index[1]
    h = x
    g4 = jnp.arange(D, dtype=jnp.int32) // 32
    r32 = jnp.arange(D, dtype=jnp.int32) % 32
    colmap = g4 * 32 + jnp.where(r32 % 2 == 0, r32 // 2, 16 + r32 // 2)
    eas = [
        lax.bitcast_convert_type(
            _ea_matmul(edge_attr, edge_W[i][:, colmap],
                       edge_b[i][colmap]).reshape(E // 2, D, 2),
            jnp.int32)
        for i in range(3)
    ]
    for i in range(3):
        ea = eas[i]
        p = _edge_sc(h, ea, src, dst)
        h = _mlp(h, p, W1[i], b1[i], gamma[i], beta[i], W2[i], b2[i])
    return _readout(h, batch, Wo1, bo1, Wo2, bo2)


# speedup vs baseline: 42.7813x; 42.7813x over previous
import jax
import jax.numpy as jnp
from jax import lax
from jax.experimental import pallas as pl
from jax.experimental.pallas import tpu as pltpu
from jax.experimental.pallas import tpu_sc as plsc

N = 10000
E = 320000
D = 128
DE = 16
G = 64

NC = 2
NS = 16
NW = NC * NS
EPW = E // NW
C = 64
NCHUNK = EPW // C
CTAIL = EPW - NCHUNK * C
RPS = 624
NTAIL = N - RPS * NS


def _edge_sc_body(h_hbm, ea_hbm, src_hbm, dst_hbm, out_hbm,
                  sidx0, sidx1, didx0, didx1, didx2,
                  hr0, hr1, hr2, ea0, ea1,
                  sidxt, didxt, hrt, eat, aggr,
                  sems0, sems1, semd0, semd1, semd2,
                  semg0, semg1, semg2, semsc0, semsc1, semsc2,
                  seme0, seme1):
    c = lax.axis_index("c")
    s = lax.axis_index("s")
    wid = s * NC + c
    base = wid * EPW
    sidxs = (sidx0, sidx1)
    semss = (sems0, sems1)
    didxs = (didx0, didx1, didx2)
    semds = (semd0, semd1, semd2)
    hrs = (hr0, hr1, hr2)
    semgs = (semg0, semg1, semg2)
    semscs = (semsc0, semsc1, semsc2)
    eabufs = (ea0, ea1)
    semes = (seme0, seme1)

    def load_sidx_ea(j, u):
        pltpu.async_copy(src_hbm.at[pl.ds(base + j * C, C)], sidxs[u % 2],
                         semss[u % 2])
        pltpu.async_copy(ea_hbm.at[pl.ds(base + j * C, C)], eabufs[u % 2],
                         semes[u % 2])

    def load_didx(j, u):
        pltpu.async_copy(dst_hbm.at[pl.ds(base + j * C, C)], didxs[u % 3],
                         semds[u % 3])

    def start_gather(j, u):
        pltpu.make_async_copy(src_hbm.at[pl.ds(base + j * C, C)],
                              sidxs[u % 2], semss[u % 2]).wait()
        pltpu.async_copy(h_hbm.at[sidxs[u % 2]], hrs[u % 3], semgs[u % 3])

    def wait_scatter(u):
        pltpu.make_async_copy(hrs[u % 3], aggr.at[didxs[u % 3]],
                              semscs[u % 3]).wait()

    load_sidx_ea(0, 0)
    load_sidx_ea(1, 1)
    load_didx(0, 0)
    start_gather(0, 0)

    pltpu.sync_copy(h_hbm.at[pl.ds(s * RPS, RPS)], aggr.at[pl.ds(s * RPS, RPS)])

    @pl.when(s == NS - 1)
    def _init_tail():
        pltpu.sync_copy(h_hbm.at[pl.ds(RPS * NS, NTAIL)],
                        aggr.at[pl.ds(RPS * NS, NTAIL)])

    plsc.subcore_barrier()

    def process(j, u, guard_lo=False, last=False, skip_prefetch=False):
        hrows = hrs[u % 3]
        eabuf = eabufs[u % 2]
        didx = didxs[u % 3]

        if not last:
            if guard_lo:
                @pl.when(j >= 2)
                def _w():
                    wait_scatter(u + 1)
            else:
                wait_scatter(u + 1)
            load_didx(j + 1, u + 1)
            start_gather(j + 1, u + 1)

        pltpu.make_async_copy(ea_hbm.at[pl.ds(base + j * C, C)], eabuf,
                              semes[u % 2]).wait()
        pltpu.make_async_copy(h_hbm.at[sidxs[u % 2]], hrows,
                              semgs[u % 3]).wait()

        def row(q, carry2):
            for rr in range(2):
                r = q * 2 + rr
                for k in range(D // 32):
                    v = eabuf[r, pl.ds(k * 16, 16)]
                    ea_lo = lax.bitcast_convert_type(v << 16, jnp.float32)
                    ea_hi = lax.bitcast_convert_type(
                        v & jnp.int32(-65536), jnp.float32)
                    sl_lo = pl.ds(k * 32, 16)
                    sl_hi = pl.ds(k * 32 + 16, 16)
                    hrows[r, sl_lo] = jnp.maximum(hrows[r, sl_lo] + ea_lo,
                                                  0.0)
                    hrows[r, sl_hi] = jnp.maximum(hrows[r, sl_hi] + ea_hi,
                                                  0.0)
            return carry2

        lax.fori_loop(0, C // 2, row, 0)
        pltpu.make_async_copy(dst_hbm.at[pl.ds(base + j * C, C)], didx,
                              semds[u % 3]).wait()
        pltpu.async_copy(hrows, aggr.at[didx], semscs[u % 3], add=True)
        if not last and not skip_prefetch:
            load_sidx_ea(j + 2, u + 2)

    UNROLL = 6
    NMAIN = (NCHUNK - 6) // UNROLL * UNROLL

    def block(t, carry):
        for u in range(UNROLL):
            process(t * UNROLL + u, u, guard_lo=(u < 2))
        return carry

    lax.fori_loop(0, NMAIN // UNROLL, block, 0)
    for j in range(NMAIN, NCHUNK):
        process(j, j % UNROLL, last=(j + 1 >= NCHUNK),
                skip_prefetch=(j + 2 >= NCHUNK))
    for jj in (NCHUNK - 3, NCHUNK - 2, NCHUNK - 1):
        wait_scatter(jj)

    toff = base + NCHUNK * C
    pltpu.sync_copy(src_hbm.at[pl.ds(toff, CTAIL)], sidxt)
    pltpu.sync_copy(dst_hbm.at[pl.ds(toff, CTAIL)], didxt)
    pltpu.sync_copy(ea_hbm.at[pl.ds(toff, CTAIL)], eat)
    pltpu.async_copy(h_hbm.at[sidxt], hrt, semg0).wait()

    def trow(r, carry2):
        for k in range(D // 32):
            v = eat[r, pl.ds(k * 16, 16)]
            ea_lo = lax.bitcast_convert_type(v << 16, jnp.float32)
            ea_hi = lax.bitcast_convert_type(v & jnp.int32(-65536),
                                             jnp.float32)
            sl_lo = pl.ds(k * 32, 16)
            sl_hi = pl.ds(k * 32 + 16, 16)
            hrt[r, sl_lo] = jnp.maximum(hrt[r, sl_lo] + ea_lo, 0.0)
            hrt[r, sl_hi] = jnp.maximum(hrt[r, sl_hi] + ea_hi, 0.0)
        return carry2

    lax.fori_loop(0, CTAIL, trow, 0)
    pltpu.sync_copy(hrt, aggr.at[didxt], add=True)

    plsc.subcore_barrier()
    pltpu.sync_copy(aggr.at[pl.ds(s * RPS, RPS)],
                    out_hbm.at[c, pl.ds(s * RPS, RPS)])

    @pl.when(s == NS - 1)
    def _out_tail():
        pltpu.sync_copy(aggr.at[pl.ds(RPS * NS, NTAIL)],
                        out_hbm.at[c, pl.ds(RPS * NS, NTAIL)])


@jax.jit
def _edge_sc(h, ea, src, dst):
    mesh = plsc.VectorSubcoreMesh(core_axis_name="c", subcore_axis_name="s")
    f = pl.kernel(
        _edge_sc_body,
        out_type=jax.ShapeDtypeStruct((NC, N, D), jnp.float32),
        mesh=mesh,
        scratch_types=(
            [pltpu.VMEM((C,), jnp.int32)] * 5
            + [pltpu.VMEM((C, D), jnp.float32)] * 3
            + [pltpu.VMEM((C, D // 2), jnp.int32)] * 2
            + [pltpu.VMEM((CTAIL,), jnp.int32)] * 2
            + [pltpu.VMEM((CTAIL, D), jnp.float32)]
            + [pltpu.VMEM((CTAIL, D // 2), jnp.int32)]
            + [pltpu.VMEM_SHARED((N, D), jnp.float32)]
            + [pltpu.SemaphoreType.DMA] * 13
        ),
    )
    return f(h, ea, src, dst)


def _ea_matmul_body(a_ref, w_ref, b_ref, o_ref):
    ea = (jnp.dot(a_ref[...], w_ref[...], preferred_element_type=jnp.float32)
          + b_ref[...])
    lo = lax.bitcast_convert_type(
        ea[:, :64].astype(jnp.bfloat16).astype(jnp.float32), jnp.int32)
    hi = lax.bitcast_convert_type(
        ea[:, 64:].astype(jnp.bfloat16).astype(jnp.float32), jnp.int32)
    o_ref[...] = (hi & jnp.int32(-65536)) | lax.shift_right_logical(lo, 16)


@jax.jit
def _ea_matmul(edge_attr, W, b):
    BE = 4000
    return pl.pallas_call(
        _ea_matmul_body,
        grid=(E // BE,),
        in_specs=[
            pl.BlockSpec((BE, DE), lambda e: (e, 0)),
            pl.BlockSpec((DE, D), lambda e: (0, 0)),
            pl.BlockSpec((1, D), lambda e: (0, 0)),
        ],
        out_specs=pl.BlockSpec((BE, D // 2), lambda e: (e, 0)),
        out_shape=jax.ShapeDtypeStruct((E, D // 2), jnp.int32),
    )(edge_attr, W, b.reshape(1, D))


def _mlp_body(h_ref, p_ref, w1_ref, b1_ref, g_ref, be_ref, w2_ref, b2_ref,
              o_ref):
    z = p_ref[0] + p_ref[1] - h_ref[...]
    z1 = jnp.dot(z, w1_ref[...], preferred_element_type=jnp.float32) + b1_ref[...]
    mu = jnp.mean(z1, axis=0, keepdims=True)
    zc = z1 - mu
    var = jnp.mean(zc * zc, axis=0, keepdims=True)
    zn = zc * (g_ref[...] * lax.rsqrt(var + 1e-5)) + be_ref[...]
    z2 = jnp.maximum(zn, 0.0)
    z3 = jnp.dot(z2, w2_ref[...], preferred_element_type=jnp.float32) + b2_ref[...]
    o_ref[...] = jnp.maximum(z3, 0.0)


@jax.jit
def _mlp(h, p, W1, b1, gamma, beta, W2, b2):
    return pl.pallas_call(
        _mlp_body,
        out_shape=jax.ShapeDtypeStruct((N, D), jnp.float32),
    )(h, p, W1, b1.reshape(1, D), gamma.reshape(1, D), beta.reshape(1, D),
      W2, b2.reshape(1, D))


def _readout_body(h_ref, b_ref, wo1_ref, bo1_ref, wo2_ref, bo2_ref, o_ref):
    bvec = b_ref[...]
    gids = lax.broadcasted_iota(jnp.int32, (G, N), 0)
    onehot = (gids == bvec).astype(jnp.float32)
    sums = jnp.dot(onehot, h_ref[...], preferred_element_type=jnp.float32)
    cnt = jnp.sum(onehot, axis=1, keepdims=True)
    pooled = sums / jnp.maximum(cnt, 1.0)
    t = jnp.maximum(
        jnp.dot(pooled, wo1_ref[...], preferred_element_type=jnp.float32)
        + bo1_ref[...], 0.0)
    o_ref[...] = (
        jnp.dot(t, wo2_ref[...], preferred_element_type=jnp.float32)
        + bo2_ref[...]
    )


@jax.jit
def _readout(h, batch, Wo1, bo1, Wo2, bo2):
    return pl.pallas_call(
        _readout_body,
        out_shape=jax.ShapeDtypeStruct((G, D), jnp.float32),
    )(h, batch.reshape(1, N), Wo1, bo1.reshape(1, D), Wo2, bo2.reshape(1, D))


def kernel(x, edge_index, edge_attr, batch, edge_W, edge_b, W1, b1, gamma,
           beta, W2, b2, Wo1, bo1, Wo2, bo2):
    src = edge_index[0]
    dst = edge_index[1]
    h = x
    half = jnp.arange(D, dtype=jnp.int32) // 64
    g4 = (jnp.arange(D, dtype=jnp.int32) % 64) // 16
    i16 = jnp.arange(D, dtype=jnp.int32) % 16
    colmap = g4 * 32 + half * 16 + i16
    eas = [_ea_matmul(edge_attr, edge_W[i][:, colmap], edge_b[i][colmap])
           for i in range(3)]
    for i in range(3):
        ea = eas[i]
        p = _edge_sc(h, ea, src, dst)
        h = _mlp(h, p, W1[i], b1[i], gamma[i], beta[i], W2[i], b2[i])
    return _readout(h, batch, Wo1, bo1, Wo2, bo2)
